# baseline (speedup 1.0000x reference)
import functools

import jax
import jax.numpy as jnp
from jax import lax
from jax.experimental import pallas as pl
from jax.experimental.pallas import tpu as pltpu
from jax.experimental.pallas import tpu_sc as plsc

N = 10000
E = 160000
D = 256
H = 8
DH = 32
HALF = D // 2

NC = 2
NS = 16
NW = NC * NS

C = 80
NCHUNK = E // C
SC1_CHUNK_IT = (NCHUNK + NW - 1) // NW
ROWS_PER_TILE = N // NS
ZCOPY = 125

_INV_SQRT_DH = 1.0 / (DH ** 0.5)


def _lane_gather(vec, idx):
    return lax.gather(
        vec, idx[:, None],
        lax.GatherDimensionNumbers(offset_dims=(), collapsed_slice_dims=(0,),
                                   start_index_map=(0,)),
        (1,), mode=lax.GatherScatterMode.PROMISE_IN_BOUNDS)


def _sc_escore_body(src_hbm, dst_hbm, kh_hbm, qh_hbm, ee_hbm, out_hbm,
                    srcv0, dstv0, kb0, qb0, eb0,
                    srcv1, dstv1, kb1, qb1, eb1, sb,
                    sk0, sq0, se0, sk1, sq1, se1):
    wid = lax.axis_index("s") * NC + lax.axis_index("c")
    slots = [
        (srcv0, dstv0, kb0, qb0, eb0, sk0, sq0, se0),
        (srcv1, dstv1, kb1, qb1, eb1, sk1, sq1, se1),
    ]

    lane = lax.iota(jnp.int32, 16)
    onehot = [(lane == h).astype(jnp.float32) for h in range(H)]
    mask8 = (lane < H).astype(jnp.float32)

    def issue(j, sl):
        srcv, dstv, kb, qb, eb, sk, sq, se = slots[sl]
        cidx = wid + j * NW

        @pl.when(cidx < NCHUNK)
        def _():
            base = cidx * C
            pltpu.sync_copy(src_hbm.at[pl.ds(base, C)], srcv)
            pltpu.sync_copy(dst_hbm.at[pl.ds(base, C)], dstv)
            pltpu.async_copy(kh_hbm.at[srcv], kb, sk)
            pltpu.async_copy(qh_hbm.at[dstv], qb, sq)
            pltpu.async_copy(ee_hbm.at[pl.ds(base, C)], eb, se)

    def compute(j, sl):
        srcv, dstv, kb, qb, eb, sk, sq, se = slots[sl]
        cidx = wid + j * NW

        @pl.when(cidx < NCHUNK)
        def _():
            pltpu.make_async_copy(kh_hbm.at[srcv], kb, sk).wait()
            pltpu.make_async_copy(qh_hbm.at[dstv], qb, sq).wait()
            pltpu.make_async_copy(ee_hbm.at[pl.ds(cidx * C, C)], eb, se).wait()

            def edge_fn(e, c2):
                svec = jnp.zeros((16,), jnp.float32)
                for h in range(H):
                    p = (kb[e, pl.ds(h * 32, 32)] * qb[e, pl.ds(h * 32, 32)]
                         * eb[e, pl.ds(h * 32, 32)])
                    lo, hi = plsc.unpack(p, format=plsc.PackFormat.INTERLEAVED)
                    a = lo + hi
                    a = a + _lane_gather(a, lane ^ 8)
                    a = a + _lane_gather(a, lane ^ 4)
                    a = a + _lane_gather(a, lane ^ 2)
                    a = a + _lane_gather(a, lane ^ 1)
                    svec = svec + a * onehot[h]
                svec = jnp.exp(jnp.clip(svec * _INV_SQRT_DH, -5.0, 5.0)) * mask8
                sb[e, :] = svec
                return c2

            lax.fori_loop(0, C, edge_fn, 0)
            pltpu.sync_copy(sb, out_hbm.at[pl.ds(cidx * C, C)])

    issue(0, 0)

    def pair_fn(jj, carry):
        j0 = 2 * jj
        issue(j0 + 1, 1)
        compute(j0, 0)
        issue(j0 + 2, 0)
        compute(j0 + 1, 1)
        return carry

    lax.fori_loop(0, (SC1_CHUNK_IT + 1) // 2, pair_fn, 0)


def _sc_escore(src, dst, kh, qh, ee):
    mesh = plsc.VectorSubcoreMesh(core_axis_name="c", subcore_axis_name="s")
    f = functools.partial(
        pl.kernel, mesh=mesh,
        compiler_params=pltpu.CompilerParams(use_tc_tiling_on_sc=False, needs_layout_passes=False),
        out_type=jax.ShapeDtypeStruct((E, 16), jnp.float32),
        scratch_types=[
            pltpu.VMEM((C,), jnp.int32),
            pltpu.VMEM((C,), jnp.int32),
            pltpu.VMEM((C, D), jnp.bfloat16),
            pltpu.VMEM((C, D), jnp.bfloat16),
            pltpu.VMEM((C, D), jnp.bfloat16),
            pltpu.VMEM((C,), jnp.int32),
            pltpu.VMEM((C,), jnp.int32),
            pltpu.VMEM((C, D), jnp.bfloat16),
            pltpu.VMEM((C, D), jnp.bfloat16),
            pltpu.VMEM((C, D), jnp.bfloat16),
            pltpu.VMEM((C, 16), jnp.float32),
            pltpu.SemaphoreType.DMA,
            pltpu.SemaphoreType.DMA,
            pltpu.SemaphoreType.DMA,
            pltpu.SemaphoreType.DMA,
            pltpu.SemaphoreType.DMA,
            pltpu.SemaphoreType.DMA,
        ],
    )(_sc_escore_body)
    return f(src, dst, kh, qh, ee)


CE = 64
NCHE = E // CE
CPS = NCHE // NC
AW = D + 32


def _sc_eaggr_body(src_hbm, dst_hbm, sc_hbm, vh_hbm,
                   outa_hbm, outb_hbm,
                   srcv0, dstv0, sbuf0, vbuf0,
                   srcv1, dstv1, sbuf1, vbuf1,
                   wbuf0, dsts0, wbuf1, dsts1,
                   ctab,
                   sg0, sg1, sw0, sw1):
    cid = lax.axis_index("c")
    sid = lax.axis_index("s")
    gslots = [(srcv0, dstv0, sbuf0, vbuf0, sg0), (srcv1, dstv1, sbuf1, vbuf1, sg1)]
    sslots = [(wbuf0, dsts0, sw0), (wbuf1, dsts1, sw1)]

    zvb = jnp.zeros((32,), jnp.bfloat16)

    def zero_bufs(r, carry):
        for q in range(AW // 32):
            wbuf0[r, pl.ds(q * 32, 32)] = zvb
        return carry

    lax.fori_loop(0, CE, zero_bufs, 0)

    r0 = sid * ROWS_PER_TILE
    for b in range(ROWS_PER_TILE // CE):
        pltpu.sync_copy(wbuf0, ctab.at[pl.ds(r0 + b * CE, CE)])
    pltpu.sync_copy(wbuf0, ctab.at[pl.ds(r0 + ROWS_PER_TILE - CE, CE)])
    plsc.subcore_barrier()

    def issue(j, sl):
        srcv, dstv, sbuf, vbuf, sg = gslots[sl]
        loc = sid + j * NS

        @pl.when(loc < CPS)
        def _():
            base = (cid * CPS + loc) * CE
            pltpu.sync_copy(src_hbm.at[pl.ds(base, CE)], srcv)
            pltpu.sync_copy(dst_hbm.at[pl.ds(base, CE)], dstv)
            pltpu.sync_copy(sc_hbm.at[pl.ds(base, CE)], sbuf)
            pltpu.async_copy(vh_hbm.at[srcv], vbuf, sg)

    def compute(j, sl):
        srcv, dstv, sbuf, vbuf, sg = gslots[sl]
        wbuf, dsts, sw = sslots[sl]
        loc = sid + j * NS

        @pl.when(loc < CPS)
        def _():
            pltpu.make_async_copy(vh_hbm.at[srcv], vbuf, sg).wait()

            def edge_fn(e, c2):
                srow = sbuf[e, :]
                for k in range(H):
                    s = _lane_gather(srow, jnp.full((16,), k, jnp.int32))
                    sbf = plsc.pack(s, s, format=plsc.PackFormat.INTERLEAVED)
                    wbuf[e, pl.ds(k * 32, 32)] = vbuf[e, pl.ds(k * 32, 32)] * sbf
                wbuf[e, pl.ds(D, 32)] = plsc.pack(
                    srow, srow, format=plsc.PackFormat.INTERLEAVED)
                return c2

            lax.fori_loop(0, CE, edge_fn, 0)

            for q in range(CE // 16):
                dsts[pl.ds(q * 16, 16)] = dstv[pl.ds(q * 16, 16)]

            pltpu.async_copy(wbuf, ctab.at[dsts], sw, add=True)

    def wait_scatter(j, sl):
        wbuf, dsts, sw = sslots[sl]
        loc = sid + j * NS

        @pl.when((j >= 0) & (loc < CPS))
        def _():
            pltpu.make_async_copy(wbuf, ctab.at[dsts], sw).wait()

    issue(0, 0)
    issue(1, 1)

    def ring_fn(q, carry):
        for r in range(2):
            j = 2 * q + r
            wait_scatter(j - 2, r)
            compute(j, r)
            issue(j + 2, r)
        return carry

    lax.fori_loop(0, 41, ring_fn, 0)
    plsc.subcore_barrier()

    for b in range(ROWS_PER_TILE // ZCOPY):
        rb = r0 + b * ZCOPY

        @pl.when(cid == 0)
        def _(rb=rb):
            pltpu.sync_copy(ctab.at[pl.ds(rb, ZCOPY)], outa_hbm.at[pl.ds(rb, ZCOPY)])

        @pl.when(cid == 1)
        def _(rb=rb):
            pltpu.sync_copy(ctab.at[pl.ds(rb, ZCOPY)], outb_hbm.at[pl.ds(rb, ZCOPY)])


def _sc_eaggr(src, dst, scores, vh):
    mesh = plsc.VectorSubcoreMesh(core_axis_name="c", subcore_axis_name="s")
    f = functools.partial(
        pl.kernel, mesh=mesh,
        compiler_params=pltpu.CompilerParams(use_tc_tiling_on_sc=False, needs_layout_passes=False),
        out_type=[
            jax.ShapeDtypeStruct((N, AW), jnp.bfloat16),
            jax.ShapeDtypeStruct((N, AW), jnp.bfloat16),
        ],
        scratch_types=(
            [pltpu.VMEM((CE,), jnp.int32),
             pltpu.VMEM((CE,), jnp.int32),
             pltpu.VMEM((CE, 16), jnp.float32),
             pltpu.VMEM((CE, D), jnp.bfloat16)] * 2
            + [pltpu.VMEM((CE, AW), jnp.bfloat16),
               pltpu.VMEM((CE,), jnp.int32)] * 2
            + [pltpu.VMEM_SHARED((N, AW), jnp.bfloat16)]
            + [pltpu.SemaphoreType.DMA] * 4
        ),
    )(_sc_eaggr_body)
    return f(src, dst, scores, vh)


BN_H = 1000
BN_E = 2000
GN = N // BN_H
GE = E // BN_E


def _tc_proj_h_body(h_ref, wq_ref, wk_ref, wv_ref, qh_ref, kh_ref, vh_ref):
    x = h_ref[...]
    q = jnp.dot(x, wq_ref[...], preferred_element_type=jnp.float32)
    k = jnp.dot(x, wk_ref[...], preferred_element_type=jnp.float32)
    qh_ref[...] = q.astype(jnp.bfloat16)
    kh_ref[...] = k.astype(jnp.bfloat16)
    v = jnp.dot(x, wv_ref[...], preferred_element_type=jnp.float32)
    vh_ref[...] = v.astype(jnp.bfloat16)


def _tc_proj_h(h, wqT, wkT, wvT):
    w_spec = pl.BlockSpec((D, D), lambda i: (0, 0))
    bh = 2000
    return pl.pallas_call(
        _tc_proj_h_body,
        grid=(N // bh,),
        in_specs=[pl.BlockSpec((bh, D), lambda i: (i, 0)), w_spec, w_spec, w_spec],
        out_specs=[
            pl.BlockSpec((bh, D), lambda i: (i, 0)),
            pl.BlockSpec((bh, D), lambda i: (i, 0)),
            pl.BlockSpec((bh, D), lambda i: (i, 0)),
        ],
        out_shape=[
            jax.ShapeDtypeStruct((N, D), jnp.bfloat16),
            jax.ShapeDtypeStruct((N, D), jnp.bfloat16),
            jax.ShapeDtypeStruct((N, D), jnp.bfloat16),
        ],
    )(h, wqT, wkT, wvT)


def _tc_proj_e_body(e_ref, we_ref, ee_ref, e2_ref, ps_ref, pq_ref):
    x = e_ref[...]
    ev = jnp.dot(x, we_ref[...], preferred_element_type=jnp.float32)
    ee_ref[...] = ev.astype(jnp.bfloat16)
    t = x + ev
    e2_ref[...] = t
    ps_ref[...] = jnp.sum(t, axis=0, keepdims=True)[None]
    pq_ref[...] = jnp.sum(t * t, axis=0, keepdims=True)[None]


def _tc_proj_e(e, weT):
    return pl.pallas_call(
        _tc_proj_e_body,
        grid=(GE,),
        in_specs=[pl.BlockSpec((BN_E, D), lambda i: (i, 0)),
                  pl.BlockSpec((D, D), lambda i: (0, 0))],
        out_specs=[
            pl.BlockSpec((BN_E, D), lambda i: (i, 0)),
            pl.BlockSpec((BN_E, D), lambda i: (i, 0)),
            pl.BlockSpec((1, 1, D), lambda i: (i, 0, 0)),
            pl.BlockSpec((1, 1, D), lambda i: (i, 0, 0)),
        ],
        out_shape=[
            jax.ShapeDtypeStruct((E, D), jnp.bfloat16),
            jax.ShapeDtypeStruct((E, D), jnp.float32),
            jax.ShapeDtypeStruct((GE, 1, D), jnp.float32),
            jax.ShapeDtypeStruct((GE, 1, D), jnp.float32),
        ],
    )(e, weT)


def _tc_attn_out_body(a_ref, b_ref, h_ref, wo_ref, bo_ref, r_ref,
                      h2_ref, ps_ref, pq_ref):
    acc = a_ref[...].astype(jnp.float32) + b_ref[...].astype(jnp.float32)
    zrep = jnp.dot(acc, r_ref[...], preferred_element_type=jnp.float32) + 1e-6
    hat = acc[:, :D] / zrep
    out = jnp.dot(hat, wo_ref[...], preferred_element_type=jnp.float32)
    out = out + bo_ref[...] + h_ref[...]
    h2_ref[...] = out
    ps_ref[...] = jnp.sum(out, axis=0, keepdims=True)[None]
    pq_ref[...] = jnp.sum(out * out, axis=0, keepdims=True)[None]


def _tc_attn_out(wva, wvb, h, woT, bo2, rmat):
    return pl.pallas_call(
        _tc_attn_out_body,
        grid=(GN,),
        in_specs=[
            pl.BlockSpec((BN_H, AW), lambda i: (i, 0)),
            pl.BlockSpec((BN_H, AW), lambda i: (i, 0)),
            pl.BlockSpec((BN_H, D), lambda i: (i, 0)),
            pl.BlockSpec((D, D), lambda i: (0, 0)),
            pl.BlockSpec((1, D), lambda i: (0, 0)),
            pl.BlockSpec((AW, D), lambda i: (0, 0)),
        ],
        out_specs=[
            pl.BlockSpec((BN_H, D), lambda i: (i, 0)),
            pl.BlockSpec((1, 1, D), lambda i: (i, 0, 0)),
            pl.BlockSpec((1, 1, D), lambda i: (i, 0, 0)),
        ],
        out_shape=[
            jax.ShapeDtypeStruct((N, D), jnp.float32),
            jax.ShapeDtypeStruct((GN, 1, D), jnp.float32),
            jax.ShapeDtypeStruct((GN, 1, D), jnp.float32),
        ],
    )(wva, wvb, h, woT, bo2, rmat)


def _tc_ffn_body(h2_ref, sc_ref, sh_ref, w1_ref, b1_ref, w2_ref, b2_ref,
                 h3_ref, ps_ref, pq_ref):
    x = h2_ref[...] * sc_ref[...] + sh_ref[...]
    f = jnp.dot(x, w1_ref[...], preferred_element_type=jnp.float32) + b1_ref[...]
    f = jnp.maximum(f, 0.0)
    g = jnp.dot(f, w2_ref[...], preferred_element_type=jnp.float32) + b2_ref[...]
    y = x + g
    h3_ref[...] = y
    ps_ref[...] = jnp.sum(y, axis=0, keepdims=True)[None]
    pq_ref[...] = jnp.sum(y * y, axis=0, keepdims=True)[None]


def _tc_ffn(h2raw, sc1, sh1, w1T, b1r, w2T, b2r):
    return pl.pallas_call(
        _tc_ffn_body,
        grid=(GN,),
        in_specs=[
            pl.BlockSpec((BN_H, D), lambda i: (i, 0)),
            pl.BlockSpec((1, D), lambda i: (0, 0)),
            pl.BlockSpec((1, D), lambda i: (0, 0)),
            pl.BlockSpec((D, 2 * D), lambda i: (0, 0)),
            pl.BlockSpec((1, 2 * D), lambda i: (0, 0)),
            pl.BlockSpec((2 * D, D), lambda i: (0, 0)),
            pl.BlockSpec((1, D), lambda i: (0, 0)),
        ],
        out_specs=[
            pl.BlockSpec((BN_H, D), lambda i: (i, 0)),
            pl.BlockSpec((1, 1, D), lambda i: (i, 0, 0)),
            pl.BlockSpec((1, 1, D), lambda i: (i, 0, 0)),
        ],
        out_shape=[
            jax.ShapeDtypeStruct((N, D), jnp.float32),
            jax.ShapeDtypeStruct((GN, 1, D), jnp.float32),
            jax.ShapeDtypeStruct((GN, 1, D), jnp.float32),
        ],
    )(h2raw, sc1, sh1, w1T, b1r, w2T, b2r)


def _tc_norm_body(x_ref, sc_ref, sh_ref, o_ref):
    o_ref[...] = x_ref[...] * sc_ref[...] + sh_ref[...]


def _tc_norm(x, scale, shift, rows, blk):
    return pl.pallas_call(
        _tc_norm_body,
        grid=(rows // blk,),
        in_specs=[
            pl.BlockSpec((blk, D), lambda i: (i, 0)),
            pl.BlockSpec((1, D), lambda i: (0, 0)),
            pl.BlockSpec((1, D), lambda i: (0, 0)),
        ],
        out_specs=pl.BlockSpec((blk, D), lambda i: (i, 0)),
        out_shape=jax.ShapeDtypeStruct((rows, D), jnp.float32),
    )(x, scale, shift)


def _tc_addnorm_body(x_ref, y_ref, sc_ref, sh_ref, o_ref):
    o_ref[...] = (x_ref[...] + y_ref[...]) * sc_ref[...] + sh_ref[...]


def _tc_addnorm(x, y, scale, shift, rows, blk):
    return pl.pallas_call(
        _tc_addnorm_body,
        grid=(rows // blk,),
        in_specs=[
            pl.BlockSpec((blk, D), lambda i: (i, 0)),
            pl.BlockSpec((blk, D), lambda i: (i, 0)),
            pl.BlockSpec((1, D), lambda i: (0, 0)),
            pl.BlockSpec((1, D), lambda i: (0, 0)),
        ],
        out_specs=pl.BlockSpec((blk, D), lambda i: (i, 0)),
        out_shape=jax.ShapeDtypeStruct((rows, D), jnp.float32),
    )(x, y, scale, shift)


def _bn_scale_shift(ps, pq, count, g, b):
    mu = jnp.sum(ps, axis=(0, 1)) / count
    var = jnp.sum(pq, axis=(0, 1)) / count - mu * mu
    scale = g / jnp.sqrt(var + 1e-5)
    shift = b - mu * scale
    return scale[None, :], shift[None, :]


def kernel(h, e, edge_index, Wq, Wk, Wv, We, Wo, bo, W1, b1, W2, b2,
           bn1h_g, bn1h_b, bn1e_g, bn1e_b, bn2h_g, bn2h_b):
    src = edge_index[0]
    dst = edge_index[1]

    hrep = jnp.repeat(jnp.eye(H, dtype=jnp.float32), DH, axis=1)
    rmat = jnp.zeros((AW, D), jnp.float32).at[D + 2 * jnp.arange(H)].set(hrep)

    qh, kh, vh = _tc_proj_h(h, Wq.T, Wk.T, Wv.T)
    ee, e2pre, pse, pqe = _tc_proj_e(e, We.T)

    scores = _sc_escore(src, dst, kh, qh, ee)
    wva, wvb = _sc_eaggr(src, dst, scores, vh)

    h2raw, ps1, pq1 = _tc_attn_out(wva, wvb, h, Wo.T, bo[None, :], rmat)
    sc1, sh1 = _bn_scale_shift(ps1, pq1, N, bn1h_g, bn1h_b)

    h3raw, ps2, pq2 = _tc_ffn(h2raw, sc1, sh1, W1.T, b1[None, :], W2.T, b2[None, :])
    sc2, sh2 = _bn_scale_shift(ps2, pq2, N, bn2h_g, bn2h_b)
    h3 = _tc_norm(h3raw, sc2, sh2, N, BN_H)

    sce, she = _bn_scale_shift(pse, pqe, E, bn1e_g, bn1e_b)
    e2 = _tc_norm(e2pre, sce, she, E, BN_E)

    return (h3, e2)


# speedup vs baseline: 1.0169x; 1.0169x over previous
import functools

import jax
import jax.numpy as jnp
from jax import lax
from jax.experimental import pallas as pl
from jax.experimental.pallas import tpu as pltpu
from jax.experimental.pallas import tpu_sc as plsc

N = 10000
E = 160000
D = 256
H = 8
DH = 32
HALF = D // 2

NC = 2
NS = 16
NW = NC * NS

C = 80
NCHUNK = E // C
SC1_CHUNK_IT = (NCHUNK + NW - 1) // NW
ROWS_PER_TILE = N // NS
ZCOPY = 125

_INV_SQRT_DH = 1.0 / (DH ** 0.5)


def _lane_gather(vec, idx):
    return lax.gather(
        vec, idx[:, None],
        lax.GatherDimensionNumbers(offset_dims=(), collapsed_slice_dims=(0,),
                                   start_index_map=(0,)),
        (1,), mode=lax.GatherScatterMode.PROMISE_IN_BOUNDS)


def _sc_escore_body(src_hbm, dst_hbm, kh_hbm, qh_hbm, ee_hbm, out_hbm,
                    srcv0, dstv0, kb0, qb0, eb0,
                    srcv1, dstv1, kb1, qb1, eb1, sb,
                    sk0, sq0, se0, sk1, sq1, se1):
    wid = lax.axis_index("s") * NC + lax.axis_index("c")
    slots = [
        (srcv0, dstv0, kb0, qb0, eb0, sk0, sq0, se0),
        (srcv1, dstv1, kb1, qb1, eb1, sk1, sq1, se1),
    ]

    lane = lax.iota(jnp.int32, 16)
    onehot = [(lane == h).astype(jnp.float32) for h in range(H)]
    mask8 = (lane < H).astype(jnp.float32)

    def issue(j, sl):
        srcv, dstv, kb, qb, eb, sk, sq, se = slots[sl]
        cidx = wid + j * NW

        @pl.when(cidx < NCHUNK)
        def _():
            base = cidx * C
            pltpu.sync_copy(src_hbm.at[pl.ds(base, C)], srcv)
            pltpu.sync_copy(dst_hbm.at[pl.ds(base, C)], dstv)
            pltpu.async_copy(kh_hbm.at[srcv], kb, sk)
            pltpu.async_copy(qh_hbm.at[dstv], qb, sq)
            pltpu.async_copy(ee_hbm.at[pl.ds(base, C)], eb, se)

    def compute(j, sl):
        srcv, dstv, kb, qb, eb, sk, sq, se = slots[sl]
        cidx = wid + j * NW

        @pl.when(cidx < NCHUNK)
        def _():
            pltpu.make_async_copy(kh_hbm.at[srcv], kb, sk).wait()
            pltpu.make_async_copy(qh_hbm.at[dstv], qb, sq).wait()
            pltpu.make_async_copy(ee_hbm.at[pl.ds(cidx * C, C)], eb, se).wait()

            def edge_fn(e, c2):
                svec = jnp.zeros((16,), jnp.float32)
                for h in range(H):
                    p = (kb[e, pl.ds(h * 32, 32)] * qb[e, pl.ds(h * 32, 32)]
                         * eb[e, pl.ds(h * 32, 32)])
                    lo, hi = plsc.unpack(p, format=plsc.PackFormat.INTERLEAVED)
                    a = lo + hi
                    a = a + _lane_gather(a, lane ^ 8)
                    a = a + _lane_gather(a, lane ^ 4)
                    a = a + _lane_gather(a, lane ^ 2)
                    a = a + _lane_gather(a, lane ^ 1)
                    svec = svec + a * onehot[h]
                svec = jnp.exp(jnp.clip(svec * _INV_SQRT_DH, -5.0, 5.0)) * mask8
                sb[e, :] = svec
                return c2

            lax.fori_loop(0, C, edge_fn, 0)
            pltpu.sync_copy(sb, out_hbm.at[pl.ds(cidx * C, C)])

    issue(0, 0)

    def pair_fn(jj, carry):
        j0 = 2 * jj
        issue(j0 + 1, 1)
        compute(j0, 0)
        issue(j0 + 2, 0)
        compute(j0 + 1, 1)
        return carry

    lax.fori_loop(0, (SC1_CHUNK_IT + 1) // 2, pair_fn, 0)


def _sc_escore(src, dst, kh, qh, ee):
    mesh = plsc.VectorSubcoreMesh(core_axis_name="c", subcore_axis_name="s")
    f = functools.partial(
        pl.kernel, mesh=mesh,
        compiler_params=pltpu.CompilerParams(use_tc_tiling_on_sc=False, needs_layout_passes=False),
        out_type=jax.ShapeDtypeStruct((E, 16), jnp.float32),
        scratch_types=[
            pltpu.VMEM((C,), jnp.int32),
            pltpu.VMEM((C,), jnp.int32),
            pltpu.VMEM((C, D), jnp.bfloat16),
            pltpu.VMEM((C, D), jnp.bfloat16),
            pltpu.VMEM((C, D), jnp.bfloat16),
            pltpu.VMEM((C,), jnp.int32),
            pltpu.VMEM((C,), jnp.int32),
            pltpu.VMEM((C, D), jnp.bfloat16),
            pltpu.VMEM((C, D), jnp.bfloat16),
            pltpu.VMEM((C, D), jnp.bfloat16),
            pltpu.VMEM((C, 16), jnp.float32),
            pltpu.SemaphoreType.DMA,
            pltpu.SemaphoreType.DMA,
            pltpu.SemaphoreType.DMA,
            pltpu.SemaphoreType.DMA,
            pltpu.SemaphoreType.DMA,
            pltpu.SemaphoreType.DMA,
        ],
    )(_sc_escore_body)
    return f(src, dst, kh, qh, ee)


CE = 64
NCHE = E // CE
CPS = NCHE // NC
AW = D + 32


def _sc_eaggr_body(src_hbm, dst_hbm, sc_hbm, vh_hbm,
                   outa_hbm, outb_hbm,
                   srcv0, dstv0, sbuf0, vbuf0,
                   srcv1, dstv1, sbuf1, vbuf1,
                   wbuf0, dsts0, wbuf1, dsts1,
                   ctab,
                   sg0, sg1, sw0, sw1):
    cid = lax.axis_index("c")
    sid = lax.axis_index("s")
    gslots = [(srcv0, dstv0, sbuf0, vbuf0, sg0), (srcv1, dstv1, sbuf1, vbuf1, sg1)]
    sslots = [(wbuf0, dsts0, sw0), (wbuf1, dsts1, sw1)]

    zvb = jnp.zeros((32,), jnp.bfloat16)

    def zero_bufs(r, carry):
        for q in range(AW // 32):
            wbuf0[r, pl.ds(q * 32, 32)] = zvb
        return carry

    lax.fori_loop(0, CE, zero_bufs, 0)

    r0 = sid * ROWS_PER_TILE
    for b in range(ROWS_PER_TILE // CE):
        pltpu.sync_copy(wbuf0, ctab.at[pl.ds(r0 + b * CE, CE)])
    pltpu.sync_copy(wbuf0, ctab.at[pl.ds(r0 + ROWS_PER_TILE - CE, CE)])
    plsc.subcore_barrier()

    def issue(j, sl):
        srcv, dstv, sbuf, vbuf, sg = gslots[sl]
        loc = sid + j * NS

        @pl.when(loc < CPS)
        def _():
            base = (cid * CPS + loc) * CE
            pltpu.sync_copy(src_hbm.at[pl.ds(base, CE)], srcv)
            pltpu.sync_copy(dst_hbm.at[pl.ds(base, CE)], dstv)
            pltpu.sync_copy(sc_hbm.at[pl.ds(base, CE)], sbuf)
            pltpu.async_copy(vh_hbm.at[srcv], vbuf, sg)

    def compute(j, sl):
        srcv, dstv, sbuf, vbuf, sg = gslots[sl]
        wbuf, dsts, sw = sslots[sl]
        loc = sid + j * NS

        @pl.when(loc < CPS)
        def _():
            pltpu.make_async_copy(vh_hbm.at[srcv], vbuf, sg).wait()

            def edge_fn(e, c2):
                srow = sbuf[e, :]
                for k in range(H):
                    s = _lane_gather(srow, jnp.full((16,), k, jnp.int32))
                    sbf = plsc.pack(s, s, format=plsc.PackFormat.INTERLEAVED)
                    wbuf[e, pl.ds(k * 32, 32)] = vbuf[e, pl.ds(k * 32, 32)] * sbf
                wbuf[e, pl.ds(D, 32)] = plsc.pack(
                    srow, srow, format=plsc.PackFormat.INTERLEAVED)
                return c2

            lax.fori_loop(0, CE, edge_fn, 0)

            for q in range(CE // 16):
                dsts[pl.ds(q * 16, 16)] = dstv[pl.ds(q * 16, 16)]

            pltpu.async_copy(wbuf, ctab.at[dsts], sw, add=True)

    def wait_scatter(j, sl):
        wbuf, dsts, sw = sslots[sl]
        loc = sid + j * NS

        @pl.when((j >= 0) & (loc < CPS))
        def _():
            pltpu.make_async_copy(wbuf, ctab.at[dsts], sw).wait()

    issue(0, 0)
    issue(1, 1)

    def ring_fn(q, carry):
        for r in range(2):
            j = 2 * q + r
            wait_scatter(j - 2, r)
            compute(j, r)
            issue(j + 2, r)
        return carry

    lax.fori_loop(0, 41, ring_fn, 0)
    plsc.subcore_barrier()

    for b in range(ROWS_PER_TILE // ZCOPY):
        rb = r0 + b * ZCOPY

        @pl.when(cid == 0)
        def _(rb=rb):
            pltpu.sync_copy(ctab.at[pl.ds(rb, ZCOPY)], outa_hbm.at[pl.ds(rb, ZCOPY)])

        @pl.when(cid == 1)
        def _(rb=rb):
            pltpu.sync_copy(ctab.at[pl.ds(rb, ZCOPY)], outb_hbm.at[pl.ds(rb, ZCOPY)])


def _sc_eaggr(src, dst, scores, vh):
    mesh = plsc.VectorSubcoreMesh(core_axis_name="c", subcore_axis_name="s")
    f = functools.partial(
        pl.kernel, mesh=mesh,
        compiler_params=pltpu.CompilerParams(use_tc_tiling_on_sc=False, needs_layout_passes=False),
        out_type=[
            jax.ShapeDtypeStruct((N, AW), jnp.bfloat16),
            jax.ShapeDtypeStruct((N, AW), jnp.bfloat16),
        ],
        scratch_types=(
            [pltpu.VMEM((CE,), jnp.int32),
             pltpu.VMEM((CE,), jnp.int32),
             pltpu.VMEM((CE, 16), jnp.float32),
             pltpu.VMEM((CE, D), jnp.bfloat16)] * 2
            + [pltpu.VMEM((CE, AW), jnp.bfloat16),
               pltpu.VMEM((CE,), jnp.int32)] * 2
            + [pltpu.VMEM_SHARED((N, AW), jnp.bfloat16)]
            + [pltpu.SemaphoreType.DMA] * 4
        ),
    )(_sc_eaggr_body)
    return f(src, dst, scores, vh)


BN_H = 1000
BN_E = 2000
GN = N // BN_H
GE = E // BN_E


def _tc_proj_h_body(h_ref, wq_ref, wk_ref, wv_ref, qh_ref, kh_ref, vh_ref):
    x = h_ref[...]
    q = jnp.dot(x, wq_ref[...], preferred_element_type=jnp.float32)
    k = jnp.dot(x, wk_ref[...], preferred_element_type=jnp.float32)
    qh_ref[...] = q.astype(jnp.bfloat16)
    kh_ref[...] = k.astype(jnp.bfloat16)
    v = jnp.dot(x, wv_ref[...], preferred_element_type=jnp.float32)
    vh_ref[...] = v.astype(jnp.bfloat16)


def _tc_proj_h(h, wqT, wkT, wvT):
    w_spec = pl.BlockSpec((D, D), lambda i: (0, 0))
    bh = 2000
    return pl.pallas_call(
        _tc_proj_h_body,
        grid=(N // bh,),
        in_specs=[pl.BlockSpec((bh, D), lambda i: (i, 0)), w_spec, w_spec, w_spec],
        out_specs=[
            pl.BlockSpec((bh, D), lambda i: (i, 0)),
            pl.BlockSpec((bh, D), lambda i: (i, 0)),
            pl.BlockSpec((bh, D), lambda i: (i, 0)),
        ],
        out_shape=[
            jax.ShapeDtypeStruct((N, D), jnp.bfloat16),
            jax.ShapeDtypeStruct((N, D), jnp.bfloat16),
            jax.ShapeDtypeStruct((N, D), jnp.bfloat16),
        ],
    )(h, wqT, wkT, wvT)


def _tc_proj_e_body(e_ref, we_ref, ee_ref, ps_ref, pq_ref):
    x = e_ref[...]
    ev = jnp.dot(x, we_ref[...], preferred_element_type=jnp.float32)
    eb = ev.astype(jnp.bfloat16)
    ee_ref[...] = eb
    t = x + eb.astype(jnp.float32)
    ps_ref[...] = jnp.sum(t, axis=0, keepdims=True)[None]
    pq_ref[...] = jnp.sum(t * t, axis=0, keepdims=True)[None]


def _tc_proj_e(e, weT):
    return pl.pallas_call(
        _tc_proj_e_body,
        grid=(GE,),
        in_specs=[pl.BlockSpec((BN_E, D), lambda i: (i, 0)),
                  pl.BlockSpec((D, D), lambda i: (0, 0))],
        out_specs=[
            pl.BlockSpec((BN_E, D), lambda i: (i, 0)),
            pl.BlockSpec((1, 1, D), lambda i: (i, 0, 0)),
            pl.BlockSpec((1, 1, D), lambda i: (i, 0, 0)),
        ],
        out_shape=[
            jax.ShapeDtypeStruct((E, D), jnp.bfloat16),
            jax.ShapeDtypeStruct((GE, 1, D), jnp.float32),
            jax.ShapeDtypeStruct((GE, 1, D), jnp.float32),
        ],
    )(e, weT)


def _tc_attn_out_body(a_ref, b_ref, h_ref, wo_ref, bo_ref, r_ref,
                      h2_ref, ps_ref, pq_ref):
    acc = a_ref[...].astype(jnp.float32) + b_ref[...].astype(jnp.float32)
    zrep = jnp.dot(acc, r_ref[...], preferred_element_type=jnp.float32) + 1e-6
    hat = acc[:, :D] / zrep
    out = jnp.dot(hat, wo_ref[...], preferred_element_type=jnp.float32)
    out = out + bo_ref[...] + h_ref[...]
    h2_ref[...] = out
    ps_ref[...] = jnp.sum(out, axis=0, keepdims=True)[None]
    pq_ref[...] = jnp.sum(out * out, axis=0, keepdims=True)[None]


def _tc_attn_out(wva, wvb, h, woT, bo2, rmat):
    return pl.pallas_call(
        _tc_attn_out_body,
        grid=(GN,),
        in_specs=[
            pl.BlockSpec((BN_H, AW), lambda i: (i, 0)),
            pl.BlockSpec((BN_H, AW), lambda i: (i, 0)),
            pl.BlockSpec((BN_H, D), lambda i: (i, 0)),
            pl.BlockSpec((D, D), lambda i: (0, 0)),
            pl.BlockSpec((1, D), lambda i: (0, 0)),
            pl.BlockSpec((AW, D), lambda i: (0, 0)),
        ],
        out_specs=[
            pl.BlockSpec((BN_H, D), lambda i: (i, 0)),
            pl.BlockSpec((1, 1, D), lambda i: (i, 0, 0)),
            pl.BlockSpec((1, 1, D), lambda i: (i, 0, 0)),
        ],
        out_shape=[
            jax.ShapeDtypeStruct((N, D), jnp.float32),
            jax.ShapeDtypeStruct((GN, 1, D), jnp.float32),
            jax.ShapeDtypeStruct((GN, 1, D), jnp.float32),
        ],
    )(wva, wvb, h, woT, bo2, rmat)


def _tc_ffn_body(h2_ref, sc_ref, sh_ref, w1_ref, b1_ref, w2_ref, b2_ref,
                 h3_ref, ps_ref, pq_ref):
    x = h2_ref[...] * sc_ref[...] + sh_ref[...]
    f = jnp.dot(x, w1_ref[...], preferred_element_type=jnp.float32) + b1_ref[...]
    f = jnp.maximum(f, 0.0)
    g = jnp.dot(f, w2_ref[...], preferred_element_type=jnp.float32) + b2_ref[...]
    y = x + g
    h3_ref[...] = y
    ps_ref[...] = jnp.sum(y, axis=0, keepdims=True)[None]
    pq_ref[...] = jnp.sum(y * y, axis=0, keepdims=True)[None]


def _tc_ffn(h2raw, sc1, sh1, w1T, b1r, w2T, b2r):
    return pl.pallas_call(
        _tc_ffn_body,
        grid=(GN,),
        in_specs=[
            pl.BlockSpec((BN_H, D), lambda i: (i, 0)),
            pl.BlockSpec((1, D), lambda i: (0, 0)),
            pl.BlockSpec((1, D), lambda i: (0, 0)),
            pl.BlockSpec((D, 2 * D), lambda i: (0, 0)),
            pl.BlockSpec((1, 2 * D), lambda i: (0, 0)),
            pl.BlockSpec((2 * D, D), lambda i: (0, 0)),
            pl.BlockSpec((1, D), lambda i: (0, 0)),
        ],
        out_specs=[
            pl.BlockSpec((BN_H, D), lambda i: (i, 0)),
            pl.BlockSpec((1, 1, D), lambda i: (i, 0, 0)),
            pl.BlockSpec((1, 1, D), lambda i: (i, 0, 0)),
        ],
        out_shape=[
            jax.ShapeDtypeStruct((N, D), jnp.float32),
            jax.ShapeDtypeStruct((GN, 1, D), jnp.float32),
            jax.ShapeDtypeStruct((GN, 1, D), jnp.float32),
        ],
    )(h2raw, sc1, sh1, w1T, b1r, w2T, b2r)


def _tc_norm_body(x_ref, sc_ref, sh_ref, o_ref):
    o_ref[...] = x_ref[...] * sc_ref[...] + sh_ref[...]


def _tc_norm(x, scale, shift, rows, blk):
    return pl.pallas_call(
        _tc_norm_body,
        grid=(rows // blk,),
        in_specs=[
            pl.BlockSpec((blk, D), lambda i: (i, 0)),
            pl.BlockSpec((1, D), lambda i: (0, 0)),
            pl.BlockSpec((1, D), lambda i: (0, 0)),
        ],
        out_specs=pl.BlockSpec((blk, D), lambda i: (i, 0)),
        out_shape=jax.ShapeDtypeStruct((rows, D), jnp.float32),
    )(x, scale, shift)


def _tc_addnorm_body(x_ref, y_ref, sc_ref, sh_ref, o_ref):
    o_ref[...] = ((x_ref[...] + y_ref[...].astype(jnp.float32))
                  * sc_ref[...] + sh_ref[...])


def _tc_addnorm(x, y, scale, shift, rows, blk):
    return pl.pallas_call(
        _tc_addnorm_body,
        grid=(rows // blk,),
        in_specs=[
            pl.BlockSpec((blk, D), lambda i: (i, 0)),
            pl.BlockSpec((blk, D), lambda i: (i, 0)),
            pl.BlockSpec((1, D), lambda i: (0, 0)),
            pl.BlockSpec((1, D), lambda i: (0, 0)),
        ],
        out_specs=pl.BlockSpec((blk, D), lambda i: (i, 0)),
        out_shape=jax.ShapeDtypeStruct((rows, D), jnp.float32),
    )(x, y, scale, shift)


def _bn_scale_shift(ps, pq, count, g, b):
    mu = jnp.sum(ps, axis=(0, 1)) / count
    var = jnp.sum(pq, axis=(0, 1)) / count - mu * mu
    scale = g / jnp.sqrt(var + 1e-5)
    shift = b - mu * scale
    return scale[None, :], shift[None, :]


def kernel(h, e, edge_index, Wq, Wk, Wv, We, Wo, bo, W1, b1, W2, b2,
           bn1h_g, bn1h_b, bn1e_g, bn1e_b, bn2h_g, bn2h_b):
    src = edge_index[0]
    dst = edge_index[1]

    hrep = jnp.repeat(jnp.eye(H, dtype=jnp.float32), DH, axis=1)
    rmat = jnp.zeros((AW, D), jnp.float32).at[D + 2 * jnp.arange(H)].set(hrep)

    qh, kh, vh = _tc_proj_h(h, Wq.T, Wk.T, Wv.T)
    ee, pse, pqe = _tc_proj_e(e, We.T)

    scores = _sc_escore(src, dst, kh, qh, ee)
    wva, wvb = _sc_eaggr(src, dst, scores, vh)

    h2raw, ps1, pq1 = _tc_attn_out(wva, wvb, h, Wo.T, bo[None, :], rmat)
    sc1, sh1 = _bn_scale_shift(ps1, pq1, N, bn1h_g, bn1h_b)

    h3raw, ps2, pq2 = _tc_ffn(h2raw, sc1, sh1, W1.T, b1[None, :], W2.T, b2[None, :])
    sc2, sh2 = _bn_scale_shift(ps2, pq2, N, bn2h_g, bn2h_b)
    h3 = _tc_norm(h3raw, sc2, sh2, N, BN_H)

    sce, she = _bn_scale_shift(pse, pqe, E, bn1e_g, bn1e_b)
    e2 = _tc_addnorm(e, ee, sce, she, E, BN_E)

    return (h3, e2)


# speedup vs baseline: 1.0301x; 1.0130x over previous
import functools

import jax
import jax.numpy as jnp
from jax import lax
from jax.experimental import pallas as pl
from jax.experimental.pallas import tpu as pltpu
from jax.experimental.pallas import tpu_sc as plsc

N = 10000
E = 160000
D = 256
H = 8
DH = 32
HALF = D // 2

NC = 2
NS = 16
NW = NC * NS

C = 128
NCHUNK = E // C
SC1_CHUNK_IT = (NCHUNK + NW - 1) // NW
ROWS_PER_TILE = N // NS
ZCOPY = 125

_INV_SQRT_DH = 1.0 / (DH ** 0.5)


def _lane_gather(vec, idx):
    return lax.gather(
        vec, idx[:, None],
        lax.GatherDimensionNumbers(offset_dims=(), collapsed_slice_dims=(0,),
                                   start_index_map=(0,)),
        (1,), mode=lax.GatherScatterMode.PROMISE_IN_BOUNDS)


def _sc_escore_body(src_hbm, dst_hbm, kh_hbm, qh_hbm, ee_hbm, out_hbm,
                    srcv0, dstv0, kb0, qb0, eb0,
                    srcv1, dstv1, kb1, qb1, eb1, sb,
                    sk0, sq0, se0, sk1, sq1, se1):
    wid = lax.axis_index("s") * NC + lax.axis_index("c")
    slots = [
        (srcv0, dstv0, kb0, qb0, eb0, sk0, sq0, se0),
        (srcv1, dstv1, kb1, qb1, eb1, sk1, sq1, se1),
    ]

    lane = lax.iota(jnp.int32, 16)
    onehot = [(lane == h).astype(jnp.float32) for h in range(H)]
    mask8 = (lane < H).astype(jnp.float32)

    def issue(j, sl):
        srcv, dstv, kb, qb, eb, sk, sq, se = slots[sl]
        cidx = wid + j * NW

        @pl.when(cidx < NCHUNK)
        def _():
            base = cidx * C
            pltpu.sync_copy(src_hbm.at[pl.ds(base, C)], srcv)
            pltpu.sync_copy(dst_hbm.at[pl.ds(base, C)], dstv)
            pltpu.async_copy(kh_hbm.at[srcv], kb, sk)
            pltpu.async_copy(qh_hbm.at[dstv], qb, sq)
            pltpu.async_copy(ee_hbm.at[pl.ds(base, C)], eb, se)

    def compute(j, sl):
        srcv, dstv, kb, qb, eb, sk, sq, se = slots[sl]
        cidx = wid + j * NW

        @pl.when(cidx < NCHUNK)
        def _():
            pltpu.make_async_copy(kh_hbm.at[srcv], kb, sk).wait()
            pltpu.make_async_copy(qh_hbm.at[dstv], qb, sq).wait()
            pltpu.make_async_copy(ee_hbm.at[pl.ds(cidx * C, C)], eb, se).wait()

            def edge_fn(e, c2):
                svec = jnp.zeros((16,), jnp.float32)
                for h in range(H):
                    p = (kb[e, pl.ds(h * 32, 32)] * qb[e, pl.ds(h * 32, 32)]
                         * eb[e, pl.ds(h * 32, 32)])
                    lo, hi = plsc.unpack(p, format=plsc.PackFormat.INTERLEAVED)
                    a = lo + hi
                    a = a + _lane_gather(a, lane ^ 8)
                    a = a + _lane_gather(a, lane ^ 4)
                    a = a + _lane_gather(a, lane ^ 2)
                    a = a + _lane_gather(a, lane ^ 1)
                    svec = svec + a * onehot[h]
                svec = jnp.exp(jnp.clip(svec * _INV_SQRT_DH, -5.0, 5.0)) * mask8
                sb[e, :] = svec
                return c2

            lax.fori_loop(0, C, edge_fn, 0)
            pltpu.sync_copy(sb, out_hbm.at[pl.ds(cidx * C, C)])

    issue(0, 0)

    def pair_fn(jj, carry):
        j0 = 2 * jj
        issue(j0 + 1, 1)
        compute(j0, 0)
        issue(j0 + 2, 0)
        compute(j0 + 1, 1)
        return carry

    lax.fori_loop(0, (SC1_CHUNK_IT + 1) // 2, pair_fn, 0)


def _sc_escore(src, dst, kh, qh, ee):
    mesh = plsc.VectorSubcoreMesh(core_axis_name="c", subcore_axis_name="s")
    f = functools.partial(
        pl.kernel, mesh=mesh,
        compiler_params=pltpu.CompilerParams(use_tc_tiling_on_sc=False, needs_layout_passes=False),
        out_type=jax.ShapeDtypeStruct((E, 16), jnp.float32),
        scratch_types=[
            pltpu.VMEM((C,), jnp.int32),
            pltpu.VMEM((C,), jnp.int32),
            pltpu.VMEM((C, D), jnp.bfloat16),
            pltpu.VMEM((C, D), jnp.bfloat16),
            pltpu.VMEM((C, D), jnp.bfloat16),
            pltpu.VMEM((C,), jnp.int32),
            pltpu.VMEM((C,), jnp.int32),
            pltpu.VMEM((C, D), jnp.bfloat16),
            pltpu.VMEM((C, D), jnp.bfloat16),
            pltpu.VMEM((C, D), jnp.bfloat16),
            pltpu.VMEM((C, 16), jnp.float32),
            pltpu.SemaphoreType.DMA,
            pltpu.SemaphoreType.DMA,
            pltpu.SemaphoreType.DMA,
            pltpu.SemaphoreType.DMA,
            pltpu.SemaphoreType.DMA,
            pltpu.SemaphoreType.DMA,
        ],
    )(_sc_escore_body)
    return f(src, dst, kh, qh, ee)


CE = 64
NCHE = E // CE
CPS = NCHE // NC
AW = D + 32


def _sc_eaggr_body(src_hbm, dst_hbm, sc_hbm, vh_hbm,
                   outa_hbm, outb_hbm,
                   srcv0, dstv0, sbuf0, vbuf0,
                   srcv1, dstv1, sbuf1, vbuf1,
                   wbuf0, dsts0, wbuf1, dsts1,
                   ctab,
                   sg0, sg1, sw0, sw1):
    cid = lax.axis_index("c")
    sid = lax.axis_index("s")
    gslots = [(srcv0, dstv0, sbuf0, vbuf0, sg0), (srcv1, dstv1, sbuf1, vbuf1, sg1)]
    sslots = [(wbuf0, dsts0, sw0), (wbuf1, dsts1, sw1)]

    zvb = jnp.zeros((32,), jnp.bfloat16)

    def zero_bufs(r, carry):
        for q in range(AW // 32):
            wbuf0[r, pl.ds(q * 32, 32)] = zvb
        return carry

    lax.fori_loop(0, CE, zero_bufs, 0)

    r0 = sid * ROWS_PER_TILE
    for b in range(ROWS_PER_TILE // CE):
        pltpu.sync_copy(wbuf0, ctab.at[pl.ds(r0 + b * CE, CE)])
    pltpu.sync_copy(wbuf0, ctab.at[pl.ds(r0 + ROWS_PER_TILE - CE, CE)])
    plsc.subcore_barrier()

    def issue(j, sl):
        srcv, dstv, sbuf, vbuf, sg = gslots[sl]
        loc = sid + j * NS

        @pl.when(loc < CPS)
        def _():
            base = (cid * CPS + loc) * CE
            pltpu.sync_copy(src_hbm.at[pl.ds(base, CE)], srcv)
            pltpu.sync_copy(dst_hbm.at[pl.ds(base, CE)], dstv)
            pltpu.sync_copy(sc_hbm.at[pl.ds(base, CE)], sbuf)
            pltpu.async_copy(vh_hbm.at[srcv], vbuf, sg)

    def compute(j, sl):
        srcv, dstv, sbuf, vbuf, sg = gslots[sl]
        wbuf, dsts, sw = sslots[sl]
        loc = sid + j * NS

        @pl.when(loc < CPS)
        def _():
            pltpu.make_async_copy(vh_hbm.at[srcv], vbuf, sg).wait()

            def edge_fn(e, c2):
                srow = sbuf[e, :]
                for k in range(H):
                    s = _lane_gather(srow, jnp.full((16,), k, jnp.int32))
                    sbf = plsc.pack(s, s, format=plsc.PackFormat.INTERLEAVED)
                    wbuf[e, pl.ds(k * 32, 32)] = vbuf[e, pl.ds(k * 32, 32)] * sbf
                wbuf[e, pl.ds(D, 32)] = plsc.pack(
                    srow, srow, format=plsc.PackFormat.INTERLEAVED)
                return c2

            lax.fori_loop(0, CE, edge_fn, 0)

            for q in range(CE // 16):
                dsts[pl.ds(q * 16, 16)] = dstv[pl.ds(q * 16, 16)]

            pltpu.async_copy(wbuf, ctab.at[dsts], sw, add=True)

    def wait_scatter(j, sl):
        wbuf, dsts, sw = sslots[sl]
        loc = sid + j * NS

        @pl.when((j >= 0) & (loc < CPS))
        def _():
            pltpu.make_async_copy(wbuf, ctab.at[dsts], sw).wait()

    issue(0, 0)
    issue(1, 1)

    def ring_fn(q, carry):
        for r in range(2):
            j = 2 * q + r
            wait_scatter(j - 2, r)
            compute(j, r)
            issue(j + 2, r)
        return carry

    lax.fori_loop(0, 41, ring_fn, 0)
    plsc.subcore_barrier()

    for b in range(ROWS_PER_TILE // ZCOPY):
        rb = r0 + b * ZCOPY

        @pl.when(cid == 0)
        def _(rb=rb):
            pltpu.sync_copy(ctab.at[pl.ds(rb, ZCOPY)], outa_hbm.at[pl.ds(rb, ZCOPY)])

        @pl.when(cid == 1)
        def _(rb=rb):
            pltpu.sync_copy(ctab.at[pl.ds(rb, ZCOPY)], outb_hbm.at[pl.ds(rb, ZCOPY)])


def _sc_eaggr(src, dst, scores, vh):
    mesh = plsc.VectorSubcoreMesh(core_axis_name="c", subcore_axis_name="s")
    f = functools.partial(
        pl.kernel, mesh=mesh,
        compiler_params=pltpu.CompilerParams(use_tc_tiling_on_sc=False, needs_layout_passes=False),
        out_type=[
            jax.ShapeDtypeStruct((N, AW), jnp.bfloat16),
            jax.ShapeDtypeStruct((N, AW), jnp.bfloat16),
        ],
        scratch_types=(
            [pltpu.VMEM((CE,), jnp.int32),
             pltpu.VMEM((CE,), jnp.int32),
             pltpu.VMEM((CE, 16), jnp.float32),
             pltpu.VMEM((CE, D), jnp.bfloat16)] * 2
            + [pltpu.VMEM((CE, AW), jnp.bfloat16),
               pltpu.VMEM((CE,), jnp.int32)] * 2
            + [pltpu.VMEM_SHARED((N, AW), jnp.bfloat16)]
            + [pltpu.SemaphoreType.DMA] * 4
        ),
    )(_sc_eaggr_body)
    return f(src, dst, scores, vh)


BN_H = 1000
BN_E = 2000
GN = N // BN_H
GE = E // BN_E


def _tc_proj_h_body(h_ref, wq_ref, wk_ref, wv_ref, qh_ref, kh_ref, vh_ref):
    x = h_ref[...]
    q = jnp.dot(x, wq_ref[...], preferred_element_type=jnp.float32)
    k = jnp.dot(x, wk_ref[...], preferred_element_type=jnp.float32)
    qh_ref[...] = q.astype(jnp.bfloat16)
    kh_ref[...] = k.astype(jnp.bfloat16)
    v = jnp.dot(x, wv_ref[...], preferred_element_type=jnp.float32)
    vh_ref[...] = v.astype(jnp.bfloat16)


def _tc_proj_h(h, wqT, wkT, wvT):
    w_spec = pl.BlockSpec((D, D), lambda i: (0, 0))
    bh = 2000
    return pl.pallas_call(
        _tc_proj_h_body,
        grid=(N // bh,),
        in_specs=[pl.BlockSpec((bh, D), lambda i: (i, 0)), w_spec, w_spec, w_spec],
        out_specs=[
            pl.BlockSpec((bh, D), lambda i: (i, 0)),
            pl.BlockSpec((bh, D), lambda i: (i, 0)),
            pl.BlockSpec((bh, D), lambda i: (i, 0)),
        ],
        out_shape=[
            jax.ShapeDtypeStruct((N, D), jnp.bfloat16),
            jax.ShapeDtypeStruct((N, D), jnp.bfloat16),
            jax.ShapeDtypeStruct((N, D), jnp.bfloat16),
        ],
    )(h, wqT, wkT, wvT)


def _tc_proj_e_body(e_ref, we_ref, ee_ref, ps_ref, pq_ref):
    x = e_ref[...]
    ev = jnp.dot(x, we_ref[...], preferred_element_type=jnp.float32)
    eb = ev.astype(jnp.bfloat16)
    ee_ref[...] = eb
    t = x + eb.astype(jnp.float32)
    ps_ref[...] = jnp.sum(t, axis=0, keepdims=True)[None]
    pq_ref[...] = jnp.sum(t * t, axis=0, keepdims=True)[None]


def _tc_proj_e(e, weT):
    return pl.pallas_call(
        _tc_proj_e_body,
        grid=(GE,),
        in_specs=[pl.BlockSpec((BN_E, D), lambda i: (i, 0)),
                  pl.BlockSpec((D, D), lambda i: (0, 0))],
        out_specs=[
            pl.BlockSpec((BN_E, D), lambda i: (i, 0)),
            pl.BlockSpec((1, 1, D), lambda i: (i, 0, 0)),
            pl.BlockSpec((1, 1, D), lambda i: (i, 0, 0)),
        ],
        out_shape=[
            jax.ShapeDtypeStruct((E, D), jnp.bfloat16),
            jax.ShapeDtypeStruct((GE, 1, D), jnp.float32),
            jax.ShapeDtypeStruct((GE, 1, D), jnp.float32),
        ],
    )(e, weT)


def _tc_attn_out_body(a_ref, b_ref, h_ref, wo_ref, bo_ref, r_ref,
                      h2_ref, ps_ref, pq_ref):
    acc = a_ref[...].astype(jnp.float32) + b_ref[...].astype(jnp.float32)
    zrep = jnp.dot(acc, r_ref[...], preferred_element_type=jnp.float32) + 1e-6
    hat = acc[:, :D] / zrep
    out = jnp.dot(hat, wo_ref[...], preferred_element_type=jnp.float32)
    out = out + bo_ref[...] + h_ref[...]
    h2_ref[...] = out
    ps_ref[...] = jnp.sum(out, axis=0, keepdims=True)[None]
    pq_ref[...] = jnp.sum(out * out, axis=0, keepdims=True)[None]


def _tc_attn_out(wva, wvb, h, woT, bo2, rmat):
    return pl.pallas_call(
        _tc_attn_out_body,
        grid=(GN,),
        in_specs=[
            pl.BlockSpec((BN_H, AW), lambda i: (i, 0)),
            pl.BlockSpec((BN_H, AW), lambda i: (i, 0)),
            pl.BlockSpec((BN_H, D), lambda i: (i, 0)),
            pl.BlockSpec((D, D), lambda i: (0, 0)),
            pl.BlockSpec((1, D), lambda i: (0, 0)),
            pl.BlockSpec((AW, D), lambda i: (0, 0)),
        ],
        out_specs=[
            pl.BlockSpec((BN_H, D), lambda i: (i, 0)),
            pl.BlockSpec((1, 1, D), lambda i: (i, 0, 0)),
            pl.BlockSpec((1, 1, D), lambda i: (i, 0, 0)),
        ],
        out_shape=[
            jax.ShapeDtypeStruct((N, D), jnp.float32),
            jax.ShapeDtypeStruct((GN, 1, D), jnp.float32),
            jax.ShapeDtypeStruct((GN, 1, D), jnp.float32),
        ],
    )(wva, wvb, h, woT, bo2, rmat)


def _tc_ffn_body(h2_ref, sc_ref, sh_ref, w1_ref, b1_ref, w2_ref, b2_ref,
                 h3_ref, ps_ref, pq_ref):
    x = h2_ref[...] * sc_ref[...] + sh_ref[...]
    f = jnp.dot(x, w1_ref[...], preferred_element_type=jnp.float32) + b1_ref[...]
    f = jnp.maximum(f, 0.0)
    g = jnp.dot(f, w2_ref[...], preferred_element_type=jnp.float32) + b2_ref[...]
    y = x + g
    h3_ref[...] = y
    ps_ref[...] = jnp.sum(y, axis=0, keepdims=True)[None]
    pq_ref[...] = jnp.sum(y * y, axis=0, keepdims=True)[None]


def _tc_ffn(h2raw, sc1, sh1, w1T, b1r, w2T, b2r):
    return pl.pallas_call(
        _tc_ffn_body,
        grid=(GN,),
        in_specs=[
            pl.BlockSpec((BN_H, D), lambda i: (i, 0)),
            pl.BlockSpec((1, D), lambda i: (0, 0)),
            pl.BlockSpec((1, D), lambda i: (0, 0)),
            pl.BlockSpec((D, 2 * D), lambda i: (0, 0)),
            pl.BlockSpec((1, 2 * D), lambda i: (0, 0)),
            pl.BlockSpec((2 * D, D), lambda i: (0, 0)),
            pl.BlockSpec((1, D), lambda i: (0, 0)),
        ],
        out_specs=[
            pl.BlockSpec((BN_H, D), lambda i: (i, 0)),
            pl.BlockSpec((1, 1, D), lambda i: (i, 0, 0)),
            pl.BlockSpec((1, 1, D), lambda i: (i, 0, 0)),
        ],
        out_shape=[
            jax.ShapeDtypeStruct((N, D), jnp.float32),
            jax.ShapeDtypeStruct((GN, 1, D), jnp.float32),
            jax.ShapeDtypeStruct((GN, 1, D), jnp.float32),
        ],
    )(h2raw, sc1, sh1, w1T, b1r, w2T, b2r)


def _tc_norm_body(x_ref, sc_ref, sh_ref, o_ref):
    o_ref[...] = x_ref[...] * sc_ref[...] + sh_ref[...]


def _tc_norm(x, scale, shift, rows, blk):
    return pl.pallas_call(
        _tc_norm_body,
        grid=(rows // blk,),
        in_specs=[
            pl.BlockSpec((blk, D), lambda i: (i, 0)),
            pl.BlockSpec((1, D), lambda i: (0, 0)),
            pl.BlockSpec((1, D), lambda i: (0, 0)),
        ],
        out_specs=pl.BlockSpec((blk, D), lambda i: (i, 0)),
        out_shape=jax.ShapeDtypeStruct((rows, D), jnp.float32),
    )(x, scale, shift)


def _tc_addnorm_body(x_ref, y_ref, sc_ref, sh_ref, o_ref):
    o_ref[...] = ((x_ref[...] + y_ref[...].astype(jnp.float32))
                  * sc_ref[...] + sh_ref[...])


def _tc_addnorm(x, y, scale, shift, rows, blk):
    return pl.pallas_call(
        _tc_addnorm_body,
        grid=(rows // blk,),
        in_specs=[
            pl.BlockSpec((blk, D), lambda i: (i, 0)),
            pl.BlockSpec((blk, D), lambda i: (i, 0)),
            pl.BlockSpec((1, D), lambda i: (0, 0)),
            pl.BlockSpec((1, D), lambda i: (0, 0)),
        ],
        out_specs=pl.BlockSpec((blk, D), lambda i: (i, 0)),
        out_shape=jax.ShapeDtypeStruct((rows, D), jnp.float32),
    )(x, y, scale, shift)


def _bn_scale_shift(ps, pq, count, g, b):
    mu = jnp.sum(ps, axis=(0, 1)) / count
    var = jnp.sum(pq, axis=(0, 1)) / count - mu * mu
    scale = g / jnp.sqrt(var + 1e-5)
    shift = b - mu * scale
    return scale[None, :], shift[None, :]


def kernel(h, e, edge_index, Wq, Wk, Wv, We, Wo, bo, W1, b1, W2, b2,
           bn1h_g, bn1h_b, bn1e_g, bn1e_b, bn2h_g, bn2h_b):
    src = edge_index[0]
    dst = edge_index[1]

    hrep = jnp.repeat(jnp.eye(H, dtype=jnp.float32), DH, axis=1)
    rmat = jnp.zeros((AW, D), jnp.float32).at[D + 2 * jnp.arange(H)].set(hrep)

    qh, kh, vh = _tc_proj_h(h, Wq.T, Wk.T, Wv.T)
    ee, pse, pqe = _tc_proj_e(e, We.T)

    scores = _sc_escore(src, dst, kh, qh, ee)
    wva, wvb = _sc_eaggr(src, dst, scores, vh)

    h2raw, ps1, pq1 = _tc_attn_out(wva, wvb, h, Wo.T, bo[None, :], rmat)
    sc1, sh1 = _bn_scale_shift(ps1, pq1, N, bn1h_g, bn1h_b)

    h3raw, ps2, pq2 = _tc_ffn(h2raw, sc1, sh1, W1.T, b1[None, :], W2.T, b2[None, :])
    sc2, sh2 = _bn_scale_shift(ps2, pq2, N, bn2h_g, bn2h_b)
    h3 = _tc_norm(h3raw, sc2, sh2, N, BN_H)

    sce, she = _bn_scale_shift(pse, pqe, E, bn1e_g, bn1e_b)
    e2 = _tc_addnorm(e, ee, sce, she, E, BN_E)

    return (h3, e2)
